# baseline (speedup 1.0000x reference)
import functools

import jax
import jax.numpy as jnp
from jax import lax
from jax.experimental import pallas as pl
from jax.experimental.pallas import tpu as pltpu
from jax.experimental.pallas import tpu_sc as plsc

_H = 64
_B = 4096
_S = 200
_NC = 2
_NS = 16
_NW = _NC * _NS
_BC = _B // _NW
_LANES = 16
_VPB = _BC // _LANES


@functools.partial(
    pl.kernel,
    out_type=jax.ShapeDtypeStruct((_S, _H, _B), jnp.float32),
    mesh=plsc.VectorSubcoreMesh(core_axis_name="c", subcore_axis_name="s"),
    compiler_params=pltpu.CompilerParams(
        use_tc_tiling_on_sc=True, needs_layout_passes=False
    ),
    scratch_types=[
        pltpu.VMEM((_S, _BC), jnp.int32),
        pltpu.VMEM((_BC, _H), jnp.float32),
        pltpu.VMEM((_BC, _H), jnp.float32),
        pltpu.VMEM((_H, _BC), jnp.float32),
        pltpu.VMEM((_H, _BC), jnp.float32),
        pltpu.VMEM((_S, _H), jnp.float32),
        pltpu.SemaphoreType.DMA((2,)),
        pltpu.SemaphoreType.DMA((2,)),
    ],
)
def _emb_kernel(ids_hbm, word_hbm, pos_hbm, out_hbm, idx_v,
                stg0, stg1, ob0, ob1, posv, sem_g, sem_w):
    stgs = [stg0, stg1]
    obs = [ob0, ob1]
    wid = lax.axis_index("s") * _NC + lax.axis_index("c")
    b0 = pl.multiple_of(wid * _BC, _BC)

    pltpu.sync_copy(ids_hbm.at[:, pl.ds(b0, _BC)], idx_v)
    pltpu.sync_copy(pos_hbm, posv)

    iota16 = jax.lax.broadcasted_iota(jnp.int32, (_LANES,), 0)

    def fire_gathers(s, p):
        stg = stgs[p]
        for j in range(_VPB):
            v = idx_v[s, pl.ds(j * _LANES, _LANES)]
            for k in range(_LANES):
                pltpu.async_copy(
                    word_hbm.at[v[k]], stg.at[j * _LANES + k], sem_g.at[p]
                )

    def wait_gathers(p):
        pltpu.make_async_copy(
            word_hbm.at[pl.ds(0, _BC)], stgs[p], sem_g.at[p]
        ).wait()

    def wait_writeback(p):
        pltpu.make_async_copy(
            obs[p], out_hbm.at[0, :, pl.ds(0, _BC)], sem_w.at[p]
        ).wait()

    def transpose_add(s, p):
        stg = stgs[p]
        ob = obs[p]
        srow = jnp.full((_LANES,), s, dtype=jnp.int32)
        for h in range(_H):
            hcol = jnp.full((_LANES,), h, dtype=jnp.int32)
            splat = plsc.load_gather(posv, [srow, hcol])
            for j in range(_VPB):
                rows = iota16 + (j * _LANES)
                vals = plsc.load_gather(stg, [rows, hcol])
                ob[h, pl.ds(j * _LANES, _LANES)] = vals + splat

    fire_gathers(0, 0)

    def iter_body(i, carry):
        for p in range(2):
            s = i * 2 + p
            wait_gathers(p)
            if p == 0:
                fire_gathers(s + 1, 1)
            else:
                @pl.when(i < _S // 2 - 1)
                def _():
                    fire_gathers(s + 1, 0)

            @pl.when(i >= 1)
            def _():
                wait_writeback(p)

            transpose_add(s, p)
            pltpu.async_copy(
                obs[p], out_hbm.at[s, :, pl.ds(b0, _BC)], sem_w.at[p]
            )
        return carry

    lax.fori_loop(0, _S // 2, iter_body, 0)

    for p in range(2):
        wait_writeback(p)


def kernel(input_ids, word_table, pos_table):
    ids_t = input_ids.T.astype(jnp.int32)
    out_t = _emb_kernel(ids_t, word_table, pos_table)
    return jnp.transpose(out_t, (2, 0, 1))


# speedup vs baseline: 2.2653x; 2.2653x over previous
import functools

import jax
import jax.numpy as jnp
from jax import lax
from jax.experimental import pallas as pl
from jax.experimental.pallas import tpu as pltpu
from jax.experimental.pallas import tpu_sc as plsc

_H = 64
_B = 4096
_S = 200
_N = _B * _S
_NC = 2
_NS = 16
_NW = _NC * _NS
_PER_W = _N // _NW
_GROUP = 128
_GROUPS = _PER_W // _GROUP
_IDX_ROWS = _N // _GROUP
_NBUF = 4
_ITERS = _GROUPS // _NBUF
_LANES = 16


@functools.partial(
    pl.kernel,
    out_type=jax.ShapeDtypeStruct((_N, _H), jnp.float32),
    mesh=plsc.VectorSubcoreMesh(core_axis_name="c", subcore_axis_name="s"),
    compiler_params=pltpu.CompilerParams(use_tc_tiling_on_sc=True),
    scratch_types=[
        pltpu.VMEM((_GROUPS, _GROUP), jnp.int32),
        pltpu.VMEM((_GROUP, _H), jnp.float32),
        pltpu.VMEM((_GROUP, _H), jnp.float32),
        pltpu.VMEM((_GROUP, _H), jnp.float32),
        pltpu.VMEM((_GROUP, _H), jnp.float32),
        pltpu.VMEM((_S, _H), jnp.float32),
        pltpu.SemaphoreType.DMA((_NBUF,)),
        pltpu.SemaphoreType.DMA((_NBUF,)),
    ],
)
def _emb_kernel(ids_hbm, word_hbm, pos_hbm, out_hbm, idx_v,
                b0, b1, b2, b3, posv, sem_g, sem_w):
    bufs = [b0, b1, b2, b3]
    wid = lax.axis_index("s") * _NC + lax.axis_index("c")
    row_base = wid * _PER_W

    pltpu.sync_copy(ids_hbm.at[pl.ds(wid * _GROUPS, _GROUPS)], idx_v)
    pltpu.sync_copy(pos_hbm, posv)

    def fire_gathers(gg, p):
        buf = bufs[p]
        for k in range(_GROUP // _LANES):
            v = idx_v[gg, pl.ds(k * _LANES, _LANES)]
            for j in range(_LANES):
                pltpu.async_copy(
                    word_hbm.at[v[j]], buf.at[k * _LANES + j], sem_g.at[p]
                )

    def wait_gathers(p):
        pltpu.make_async_copy(
            out_hbm.at[pl.ds(0, _GROUP)], bufs[p], sem_g.at[p]
        ).wait()

    def wait_writeback(p):
        pltpu.make_async_copy(
            bufs[p], out_hbm.at[pl.ds(0, _GROUP)], sem_w.at[p]
        ).wait()

    def add_pos(gg, p):
        buf = bufs[p]
        pbase = lax.rem(gg * _GROUP, _S)

        @plsc.parallel_loop(0, _GROUP, unroll=2)
        def _(r):
            s = pbase + r
            s = jnp.where(s >= _S, s - _S, s)
            for c in range(_H // _LANES):
                sl = pl.ds(c * _LANES, _LANES)
                buf[r, sl] = buf[r, sl] + posv[s, sl]

    for p in range(_NBUF - 1):
        fire_gathers(p, p)

    def iter_body(i, carry):
        for p in range(_NBUF):
            gg = i * _NBUF + p
            nxt = (p + _NBUF - 1) % _NBUF
            if p == 0:
                @pl.when(i >= 1)
                def _():
                    wait_writeback(nxt)
                fire_gathers(gg + _NBUF - 1, nxt)
            else:
                @pl.when(i < _ITERS - 1)
                def _():
                    wait_writeback(nxt)
                    fire_gathers(gg + _NBUF - 1, nxt)

            wait_gathers(p)
            add_pos(gg, p)
            pltpu.async_copy(
                bufs[p], out_hbm.at[pl.ds(row_base + gg * _GROUP, _GROUP)],
                sem_w.at[p],
            )
        return carry

    lax.fori_loop(0, _ITERS, iter_body, 0)

    for p in range(_NBUF):
        wait_writeback(p)


def kernel(input_ids, word_table, pos_table):
    ids = input_ids.reshape(_IDX_ROWS, _GROUP).astype(jnp.int32)
    out = _emb_kernel(ids, word_table, pos_table)
    return out.reshape(_B, _S, _H)
